# baseline (speedup 1.0000x reference)
import functools
import math

import jax
import jax.numpy as jnp
from jax.experimental import pallas as pl
from jax.experimental.pallas import tpu as pltpu

NUM_CLASSES = 8
NUM_ANCHORS = 3
NC5 = 5 + NUM_CLASSES
VMEM_LIMIT = 64 * 1024 * 1024


def _conv1_head_kernel(x_ref, w_ref, b_ref, ws_ref, bs_ref, f_ref, smax_ref,
                       *, th, wo, n_chunks):
    m = jnp.zeros((), jnp.float32)
    for c in range(n_chunks):
        patch = x_ref[pl.ds(c * th, th), :, :].reshape(th * wo, 108)
        y = jnp.dot(patch, w_ref[...], preferred_element_type=jnp.float32)
        y = y + b_ref[...]
        y = y * (1.0 / (1.0 + jnp.exp(-y)))
        ybf = y.astype(jnp.bfloat16)
        f_ref[pl.ds(c * th * wo, th * wo), :] = ybf
        z = jnp.dot(ybf, ws_ref[...], preferred_element_type=jnp.float32)
        s = 1.0 / (1.0 + jnp.exp(-(z + bs_ref[...])))
        p = s[:, 0:64] * s[:, 64:128]
        m = jnp.maximum(m, jnp.max(p))
    smax_ref[...] = m * jnp.ones_like(smax_ref)


def _convN_head_kernel(x_ref, w_ref, b_ref, ws_ref, bs_ref, *o_refs,
                       th, wo, cin, n_chunks, write_feat):
    if write_feat:
        f_ref, smax_ref = o_refs
    else:
        (smax_ref,) = o_refs
    m = jnp.zeros((), jnp.float32)
    for c in range(n_chunks):
        r0 = c * th
        acc = jnp.zeros((th * wo, w_ref.shape[-1]), jnp.float32)
        for ki in range(3):
            for kj in range(3):
                win = x_ref[pl.ds(r0 + ki, th), pl.ds(kj, wo), :]
                patch = win.reshape(th * wo, cin)
                acc = acc + jnp.dot(patch, w_ref[ki * 3 + kj],
                                    preferred_element_type=jnp.float32)
        y = acc + b_ref[...]
        y = y * (1.0 / (1.0 + jnp.exp(-y)))
        ybf = y.astype(jnp.bfloat16)
        if write_feat:
            f_ref[pl.ds(c * th * wo, th * wo), :] = ybf
        z = jnp.dot(ybf, ws_ref[...], preferred_element_type=jnp.float32)
        s = 1.0 / (1.0 + jnp.exp(-(z + bs_ref[...])))
        p = s[:, 0:64] * s[:, 64:128]
        m = jnp.maximum(m, jnp.max(p))
    smax_ref[...] = m * jnp.ones_like(smax_ref)


def _conv1_level(xcat, w108, b, wsel, bsel, *, h, wo):
    bsz = xcat.shape[0]
    th = min(32, h)
    res = pl.pallas_call(
        functools.partial(_conv1_head_kernel, th=th, wo=wo, n_chunks=h // th),
        grid=(bsz,),
        in_specs=[
            pl.BlockSpec((None, h, wo, 108), lambda bi: (bi, 0, 0, 0)),
            pl.BlockSpec((108, 64), lambda bi: (0, 0)),
            pl.BlockSpec((1, 64), lambda bi: (0, 0)),
            pl.BlockSpec((64, 128), lambda bi: (0, 0)),
            pl.BlockSpec((1, 128), lambda bi: (0, 0)),
        ],
        out_specs=[
            pl.BlockSpec((None, h * wo, 64), lambda bi: (bi, 0, 0)),
            pl.BlockSpec((None, 1, 128), lambda bi: (bi, 0, 0)),
        ],
        out_shape=[
            jax.ShapeDtypeStruct((bsz, h * wo, 64), jnp.bfloat16),
            jax.ShapeDtypeStruct((bsz, 1, 128), jnp.float32),
        ],
        compiler_params=pltpu.CompilerParams(
            dimension_semantics=("parallel",),
            vmem_limit_bytes=VMEM_LIMIT),
    )(xcat, w108, b, wsel, bsel)
    return res[0], res[1]


def _convN_level(xp, w9, b, wsel, bsel, *, h, wo, cout, write_feat):
    bsz, hp, wp, cin = xp.shape
    th = min(max(1, 512 // wo), h)
    res = pl.pallas_call(
        functools.partial(_convN_head_kernel, th=th, wo=wo, cin=cin,
                          n_chunks=h // th, write_feat=write_feat),
        grid=(bsz,),
        in_specs=[
            pl.BlockSpec((None, hp, wp, cin), lambda bi: (bi, 0, 0, 0)),
            pl.BlockSpec((9, cin, cout), lambda bi: (0, 0, 0)),
            pl.BlockSpec((1, cout), lambda bi: (0, 0)),
            pl.BlockSpec((cout, 128), lambda bi: (0, 0)),
            pl.BlockSpec((1, 128), lambda bi: (0, 0)),
        ],
        out_specs=([pl.BlockSpec((None, h * wo, cout), lambda bi: (bi, 0, 0))]
                   if write_feat else []) +
                  [pl.BlockSpec((None, 1, 128), lambda bi: (bi, 0, 0))],
        out_shape=([jax.ShapeDtypeStruct((bsz, h * wo, cout), jnp.bfloat16)]
                   if write_feat else []) +
                  [jax.ShapeDtypeStruct((bsz, 1, 128), jnp.float32)],
        compiler_params=pltpu.CompilerParams(
            dimension_semantics=("parallel",),
            vmem_limit_bytes=VMEM_LIMIT),
    )(xp, w9, b, wsel, bsel)
    if write_feat:
        return res[0], res[1]
    return None, res[0]


def _loss_kernel(sm_ref, loss_ref):
    m = jnp.max(sm_ref[...])
    loss_ref[...] = -jnp.log(jnp.maximum(1.0 - m, 1e-9)) * jnp.ones_like(loss_ref)


def _space_to_depth(x):
    b, h, w, c = x.shape
    x = x.reshape(b, h // 2, 2, w // 2, 2, c)
    x = jnp.transpose(x, (0, 1, 3, 2, 4, 5))
    return x.reshape(b, h // 2, w // 2, 4 * c)


def _head_select(w, b, t):
    cin = w.shape[0]
    obj_cols = jnp.array([a * NC5 + 4 for a in range(NUM_ANCHORS)], jnp.int32)
    cls_cols = jnp.array([a * NC5 + 5 for a in range(NUM_ANCHORS)], jnp.int32) + t
    wobj = jnp.take(w, obj_cols, axis=1)
    wcls = jnp.take(w, cls_cols, axis=1)
    wsel = jnp.zeros((cin, 128), jnp.bfloat16)
    wsel = wsel.at[:, 0:3].set(wobj.astype(jnp.bfloat16))
    wsel = wsel.at[:, 64:67].set(wcls.astype(jnp.bfloat16))
    bsel = jnp.full((1, 128), -30.0, jnp.float32)
    bsel = bsel.at[0, 0:3].set(jnp.take(b[0], obj_cols))
    bsel = bsel.at[0, 64:67].set(jnp.take(b[0], cls_cols))
    return wsel, bsel


def kernel(x, attack_target, conv1_w, conv1_b, conv2_w, conv2_b, conv3_w,
           conv3_b, head1_w, head1_b, head2_w, head2_b, head3_w, head3_b):
    t = jnp.asarray(attack_target, jnp.int32)
    x = jnp.transpose(x, (0, 2, 3, 1)).astype(jnp.float32)
    bsz, h, w, _ = x.shape
    h1, w1 = h // 2, w // 2

    xs = _space_to_depth(x)
    xp = jnp.pad(xs, ((0, 0), (1, 1), (1, 1), (0, 0))).astype(jnp.bfloat16)
    xcat = jnp.concatenate(
        [xp[:, ki:ki + h1, kj:kj + w1, :] for ki in range(3) for kj in range(3)],
        axis=-1)

    ws1, bs1 = _head_select(head1_w, head1_b, t)
    ws2, bs2 = _head_select(head2_w, head2_b, t)
    ws3, bs3 = _head_select(head3_w, head3_b, t)

    f1, sm1 = _conv1_level(xcat, conv1_w.reshape(108, 64), conv1_b, ws1, bs1,
                           h=h1, wo=w1)
    f1_img = f1.reshape(bsz, h1, w1, 64)
    x2 = jnp.pad(_space_to_depth(f1_img), ((0, 0), (1, 1), (1, 1), (0, 0)))
    f2, sm2 = _convN_level(x2, conv2_w, conv2_b, ws2, bs2,
                           h=h // 4, wo=w // 4, cout=128, write_feat=True)
    f2_img = f2.reshape(bsz, h // 4, w // 4, 128)
    x3 = jnp.pad(_space_to_depth(f2_img), ((0, 0), (1, 1), (1, 1), (0, 0)))
    _, sm3 = _convN_level(x3, conv3_w, conv3_b, ws3, bs3,
                          h=h // 8, wo=w // 8, cout=128, write_feat=False)

    sm = jnp.concatenate([sm1, sm2, sm3], axis=0).reshape(-1, 128)
    loss = pl.pallas_call(
        _loss_kernel,
        grid=(1,),
        in_specs=[pl.BlockSpec((sm.shape[0], 128), lambda i: (0, 0))],
        out_specs=pl.BlockSpec((1, 1), lambda i: (0, 0)),
        out_shape=jax.ShapeDtypeStruct((1, 1), jnp.float32),
    )(sm)
    return loss[0, 0]


# speedup vs baseline: 2.6843x; 2.6843x over previous
import functools
import math

import jax
import jax.numpy as jnp
from jax.experimental import pallas as pl
from jax.experimental.pallas import tpu as pltpu

NUM_CLASSES = 8
NUM_ANCHORS = 3
NC5 = 5 + NUM_CLASSES
VMEM_LIMIT = 100 * 1024 * 1024


def _quad_conv1_weights(w):
    zero = jnp.zeros((12, 64), w.dtype)
    taps = []
    for ki in range(3):
        for dvi in range(3):
            rows = []
            for vp in range(4):
                cols = []
                for q4 in range(4):
                    kj = 4 * (dvi - 1) + vp + 1 - q4
                    cols.append(w[ki * 3 + kj] if 0 <= kj < 3 else zero)
                rows.append(jnp.concatenate(cols, axis=1))
            taps.append(jnp.concatenate(rows, axis=0))
    return jnp.stack(taps)


def _pair_conv_weights(w):
    cin, cout = w.shape[1], w.shape[2]
    zero = jnp.zeros((cin, cout), w.dtype)
    taps = []
    for di in range(3):
        for dvi in range(3):
            rows = []
            for vp in range(2):
                cols = []
                for q in range(2):
                    dj = 2 * (dvi - 1) + vp + 1 - q
                    cols.append(w[di * 3 + dj] if 0 <= dj < 3 else zero)
                rows.append(jnp.concatenate(cols, axis=1))
            taps.append(jnp.concatenate(rows, axis=0))
    return jnp.stack(taps)


def _head_select(w, b, t, nq):
    cin = w.shape[0]
    obj_cols = jnp.array([a * NC5 + 4 for a in range(NUM_ANCHORS)], jnp.int32)
    cls_cols = jnp.array([a * NC5 + 5 for a in range(NUM_ANCHORS)], jnp.int32) + t
    wobj = jnp.take(w, obj_cols, axis=1).astype(jnp.bfloat16)
    wcls = jnp.take(w, cls_cols, axis=1).astype(jnp.bfloat16)
    ws = jnp.zeros((nq * cin, 128), jnp.bfloat16)
    bs = jnp.full((1, 128), -30.0, jnp.float32)
    for q in range(nq):
        ws = ws.at[q * cin:(q + 1) * cin, q * 16:q * 16 + 3].set(wobj)
        ws = ws.at[q * cin:(q + 1) * cin, 64 + q * 16:64 + q * 16 + 3].set(wcls)
        bs = bs.at[0, q * 16:q * 16 + 3].set(jnp.take(b[0], obj_cols))
        bs = bs.at[0, 64 + q * 16:64 + q * 16 + 3].set(jnp.take(b[0], cls_cols))
    return ws, bs


def _sigmoid(z):
    return 1.0 / (1.0 + jnp.exp(-z))


def _mega_kernel(xq_ref, w1_ref, b1_ref, h1w_ref, h1b_ref,
                 w2_ref, b2_ref, h2w_ref, h2b_ref,
                 w3_ref, b3_ref, h3w_ref, h3b_ref,
                 smax_ref, f1p_ref, f2p_ref,
                 *, h, w):
    hr1, wq1 = h // 2, w // 8
    hr2, wp2 = h // 4, w // 8
    hr3, wp3 = h // 8, w // 16

    f1p_ref[pl.ds(0, 1), :, :] = jnp.zeros((1, wp2 + 2, 512), jnp.bfloat16)
    f1p_ref[pl.ds(hr2 + 1, 1), :, :] = jnp.zeros((1, wp2 + 2, 512), jnp.bfloat16)
    f1p_ref[:, pl.ds(0, 1), :] = jnp.zeros((hr2 + 2, 1, 512), jnp.bfloat16)
    f1p_ref[:, pl.ds(wp2 + 1, 1), :] = jnp.zeros((hr2 + 2, 1, 512), jnp.bfloat16)
    f2p_ref[pl.ds(0, 1), :, :] = jnp.zeros((1, wp3 + 2, 1024), jnp.bfloat16)
    f2p_ref[pl.ds(hr3 + 1, 1), :, :] = jnp.zeros((1, wp3 + 2, 1024), jnp.bfloat16)
    f2p_ref[:, pl.ds(0, 1), :] = jnp.zeros((hr3 + 2, 1, 1024), jnp.bfloat16)
    f2p_ref[:, pl.ds(wp3 + 1, 1), :] = jnp.zeros((hr3 + 2, 1, 1024), jnp.bfloat16)

    th1 = min(8, hr1)
    n1 = hr1 // th1

    def level1(c, m):
        r0 = c * th1
        acc = jnp.zeros((th1 * wq1, 256), jnp.float32)
        for ki in range(3):
            for dvi in range(3):
                win = xq_ref[pl.ds(r0 + ki, th1), pl.ds(dvi, wq1), :]
                patch = win.reshape(th1 * wq1, 48)
                acc = acc + jnp.dot(patch, w1_ref[ki * 3 + dvi],
                                    preferred_element_type=jnp.float32)
        y = acc + b1_ref[...]
        y = y * _sigmoid(y)
        ybf = y.astype(jnp.bfloat16)
        z = jnp.dot(ybf, h1w_ref[...], preferred_element_type=jnp.float32)
        s = _sigmoid(z + h1b_ref[...])
        p = s[:, 0:64] * s[:, 64:128]
        m = jnp.maximum(m, jnp.max(p))
        yv = ybf.reshape(th1 // 2, 2, wq1, 256)
        for par in range(2):
            f1p_ref[pl.ds(1 + r0 // 2, th1 // 2), pl.ds(1, wq1),
                    pl.ds(par * 128, 128)] = yv[:, par, :, 0:128]
            f1p_ref[pl.ds(1 + r0 // 2, th1 // 2), pl.ds(1, wq1),
                    pl.ds(256 + par * 128, 128)] = yv[:, par, :, 128:256]
        return m

    m = jax.lax.fori_loop(0, n1, level1, jnp.zeros((), jnp.float32))

    th2 = min(8, hr2)
    n2 = hr2 // th2

    def level2(c, m):
        i0 = c * th2
        acc = jnp.zeros((th2 * wp2, 256), jnp.float32)
        for di in range(3):
            for dvi in range(3):
                win = f1p_ref[pl.ds(i0 + di, th2), pl.ds(dvi, wp2), :]
                patch = win.reshape(th2 * wp2, 512)
                acc = acc + jnp.dot(patch, w2_ref[di * 3 + dvi],
                                    preferred_element_type=jnp.float32)
        y = acc + b2_ref[...]
        y = y * _sigmoid(y)
        ybf = y.astype(jnp.bfloat16)
        z = jnp.dot(ybf, h2w_ref[...], preferred_element_type=jnp.float32)
        s = _sigmoid(z + h2b_ref[...])
        p = s[:, 0:64] * s[:, 64:128]
        m = jnp.maximum(m, jnp.max(p))
        yv = ybf.reshape(th2 // 2, 2, wp2 // 2, 2, 256)
        for par in range(2):
            for vp in range(2):
                f2p_ref[pl.ds(1 + i0 // 2, th2 // 2), pl.ds(1, wp2 // 2),
                        pl.ds(vp * 512 + par * 256, 256)] = yv[:, par, :, vp, :]
        return m

    m = jax.lax.fori_loop(0, n2, level2, m)

    th3 = min(16, hr3)
    for c in range(hr3 // th3):
        i0 = c * th3
        acc = jnp.zeros((th3 * wp3, 256), jnp.float32)
        for di in range(3):
            for dvi in range(3):
                win = f2p_ref[pl.ds(i0 + di, th3), pl.ds(dvi, wp3), :]
                patch = win.reshape(th3 * wp3, 1024)
                acc = acc + jnp.dot(patch, w3_ref[di * 3 + dvi],
                                    preferred_element_type=jnp.float32)
        y = acc + b3_ref[...]
        y = y * _sigmoid(y)
        ybf = y.astype(jnp.bfloat16)
        z = jnp.dot(ybf, h3w_ref[...], preferred_element_type=jnp.float32)
        s = _sigmoid(z + h3b_ref[...])
        p = s[:, 0:64] * s[:, 64:128]
        m = jnp.maximum(m, jnp.max(p))

    smax_ref[...] = m * jnp.ones_like(smax_ref)


def _loss_kernel(sm_ref, loss_ref):
    m = jnp.max(sm_ref[...])
    loss_ref[...] = -jnp.log(jnp.maximum(1.0 - m, 1e-9)) * jnp.ones_like(loss_ref)


def _space_to_depth(x):
    b, h, w, c = x.shape
    x = x.reshape(b, h // 2, 2, w // 2, 2, c)
    x = jnp.transpose(x, (0, 1, 3, 2, 4, 5))
    return x.reshape(b, h // 2, w // 2, 4 * c)


def kernel(x, attack_target, conv1_w, conv1_b, conv2_w, conv2_b, conv3_w,
           conv3_b, head1_w, head1_b, head2_w, head2_b, head3_w, head3_b):
    t = jnp.asarray(attack_target, jnp.int32)
    x = jnp.transpose(x, (0, 2, 3, 1)).astype(jnp.float32)
    bsz, h, w, _ = x.shape

    xs = _space_to_depth(x)
    xq = xs.reshape(bsz, h // 2, w // 8, 48)
    xq = jnp.pad(xq, ((0, 0), (1, 1), (1, 1), (0, 0))).astype(jnp.bfloat16)

    w1 = _quad_conv1_weights(conv1_w)
    w2 = _pair_conv_weights(conv2_w)
    w3 = _pair_conv_weights(conv3_w)
    b1 = jnp.tile(conv1_b, (1, 4))
    b2 = jnp.tile(conv2_b, (1, 2))
    b3 = jnp.tile(conv3_b, (1, 2))
    h1w, h1b = _head_select(head1_w, head1_b, t, 4)
    h2w, h2b = _head_select(head2_w, head2_b, t, 2)
    h3w, h3b = _head_select(head3_w, head3_b, t, 2)

    const = lambda bi: (0, 0)
    const3 = lambda bi: (0, 0, 0)
    sm = pl.pallas_call(
        functools.partial(_mega_kernel, h=h, w=w),
        grid=(bsz,),
        in_specs=[
            pl.BlockSpec((None, h // 2 + 2, w // 8 + 2, 48),
                         lambda bi: (bi, 0, 0, 0)),
            pl.BlockSpec((9, 48, 256), const3),
            pl.BlockSpec((1, 256), const),
            pl.BlockSpec((256, 128), const),
            pl.BlockSpec((1, 128), const),
            pl.BlockSpec((9, 512, 256), const3),
            pl.BlockSpec((1, 256), const),
            pl.BlockSpec((256, 128), const),
            pl.BlockSpec((1, 128), const),
            pl.BlockSpec((9, 1024, 256), const3),
            pl.BlockSpec((1, 256), const),
            pl.BlockSpec((256, 128), const),
            pl.BlockSpec((1, 128), const),
        ],
        out_specs=pl.BlockSpec((None, 1, 128), lambda bi: (bi, 0, 0)),
        out_shape=jax.ShapeDtypeStruct((bsz, 1, 128), jnp.float32),
        scratch_shapes=[
            pltpu.VMEM((h // 4 + 2, w // 8 + 2, 512), jnp.bfloat16),
            pltpu.VMEM((h // 8 + 2, w // 16 + 2, 1024), jnp.bfloat16),
        ],
        compiler_params=pltpu.CompilerParams(
            dimension_semantics=("parallel",),
            vmem_limit_bytes=VMEM_LIMIT),
    )(xq, w1, b1, h1w, h1b, w2, b2, h2w, h2b, w3, b3, h3w, h3b)

    loss = pl.pallas_call(
        _loss_kernel,
        grid=(1,),
        in_specs=[pl.BlockSpec((bsz, 128), lambda i: (0, 0))],
        out_specs=pl.BlockSpec((1, 1), lambda i: (0, 0)),
        out_shape=jax.ShapeDtypeStruct((1, 1), jnp.float32),
    )(sm.reshape(bsz, 128))
    return loss[0, 0]


# speedup vs baseline: 3.2127x; 1.1969x over previous
import functools
import math

import jax
import jax.numpy as jnp
from jax.experimental import pallas as pl
from jax.experimental.pallas import tpu as pltpu

NUM_CLASSES = 8
NUM_ANCHORS = 3
NC5 = 5 + NUM_CLASSES
VMEM_LIMIT = 100 * 1024 * 1024


def _quad_conv1_weights(w):
    zero = jnp.zeros((12, 64), w.dtype)
    taps = []
    for ki in range(3):
        for dvi in range(3):
            rows = []
            for vp in range(4):
                cols = []
                for q4 in range(4):
                    kj = 4 * (dvi - 1) + vp + 1 - q4
                    cols.append(w[ki * 3 + kj] if 0 <= kj < 3 else zero)
                rows.append(jnp.concatenate(cols, axis=1))
            taps.append(jnp.concatenate(rows, axis=0))
    return jnp.stack(taps)


def _pair_conv_weights(w):
    cin, cout = w.shape[1], w.shape[2]
    zero = jnp.zeros((cin, cout), w.dtype)
    taps = []
    for di in range(3):
        for dvi in range(3):
            rows = []
            for vp in range(2):
                cols = []
                for q in range(2):
                    dj = 2 * (dvi - 1) + vp + 1 - q
                    cols.append(w[di * 3 + dj] if 0 <= dj < 3 else zero)
                rows.append(jnp.concatenate(cols, axis=1))
            taps.append(jnp.concatenate(rows, axis=0))
    return jnp.stack(taps)


def _head_select(w, b, t, nq):
    cin = w.shape[0]
    obj_cols = jnp.array([a * NC5 + 4 for a in range(NUM_ANCHORS)], jnp.int32)
    cls_cols = jnp.array([a * NC5 + 5 for a in range(NUM_ANCHORS)], jnp.int32) + t
    wobj = jnp.take(w, obj_cols, axis=1).astype(jnp.bfloat16)
    wcls = jnp.take(w, cls_cols, axis=1).astype(jnp.bfloat16)
    ws = jnp.zeros((nq * cin, 128), jnp.bfloat16)
    bs = jnp.full((1, 128), -30.0, jnp.float32)
    for q in range(nq):
        ws = ws.at[q * cin:(q + 1) * cin, q * 16:q * 16 + 3].set(wobj)
        ws = ws.at[q * cin:(q + 1) * cin, 64 + q * 16:64 + q * 16 + 3].set(wcls)
        bs = bs.at[0, q * 16:q * 16 + 3].set(jnp.take(b[0], obj_cols))
        bs = bs.at[0, 64 + q * 16:64 + q * 16 + 3].set(jnp.take(b[0], cls_cols))
    return ws, bs


def _sigmoid(z):
    return 1.0 / (1.0 + jnp.exp(-z))


def _mega_kernel(xq_ref, w1_ref, b1_ref, h1w_ref, h1b_ref,
                 w2_ref, b2_ref, h2w_ref, h2b_ref,
                 w3_ref, b3_ref, h3w_ref, h3b_ref,
                 smax_ref, f1p_ref, f2p_ref,
                 *, h, w):
    hr1, wq1 = h // 2, w // 8
    hr2, wp2 = h // 4, w // 8
    hr3, wp3 = h // 8, w // 16

    f1p_ref[pl.ds(0, 1), :, :] = jnp.zeros((1, wp2 + 2, 512), jnp.bfloat16)
    f1p_ref[pl.ds(hr2 + 1, 1), :, :] = jnp.zeros((1, wp2 + 2, 512), jnp.bfloat16)
    f1p_ref[:, pl.ds(0, 1), :] = jnp.zeros((hr2 + 2, 1, 512), jnp.bfloat16)
    f1p_ref[:, pl.ds(wp2 + 1, 1), :] = jnp.zeros((hr2 + 2, 1, 512), jnp.bfloat16)
    f2p_ref[pl.ds(0, 1), :, :] = jnp.zeros((1, wp3 + 2, 1024), jnp.bfloat16)
    f2p_ref[pl.ds(hr3 + 1, 1), :, :] = jnp.zeros((1, wp3 + 2, 1024), jnp.bfloat16)
    f2p_ref[:, pl.ds(0, 1), :] = jnp.zeros((hr3 + 2, 1, 1024), jnp.bfloat16)
    f2p_ref[:, pl.ds(wp3 + 1, 1), :] = jnp.zeros((hr3 + 2, 1, 1024), jnp.bfloat16)

    th1 = min(16, hr1)
    n1 = hr1 // th1

    def level1(c, m):
        r0 = c * th1
        acc = jnp.zeros((th1 * wq1, 256), jnp.float32)
        for ki in range(3):
            for dvi in range(3):
                win = xq_ref[pl.ds(r0 + ki, th1), pl.ds(dvi, wq1), :]
                patch = win.reshape(th1 * wq1, 48)
                acc = acc + jnp.dot(patch, w1_ref[ki * 3 + dvi],
                                    preferred_element_type=jnp.float32)
        y = acc + b1_ref[...]
        y = y * _sigmoid(y)
        ybf = y.astype(jnp.bfloat16)
        z = jnp.dot(ybf, h1w_ref[...], preferred_element_type=jnp.float32)
        s = _sigmoid(z + h1b_ref[...])
        p = s[:, 0:64] * s[:, 64:128]
        m = jnp.maximum(m, jnp.max(p))
        yv = ybf.reshape(th1 // 2, 2, wq1, 256)
        for par in range(2):
            f1p_ref[pl.ds(1 + r0 // 2, th1 // 2), pl.ds(1, wq1),
                    pl.ds(par * 128, 128)] = yv[:, par, :, 0:128]
            f1p_ref[pl.ds(1 + r0 // 2, th1 // 2), pl.ds(1, wq1),
                    pl.ds(256 + par * 128, 128)] = yv[:, par, :, 128:256]
        return m

    m = jax.lax.fori_loop(0, n1, level1, jnp.zeros((), jnp.float32))

    th2 = min(16, hr2)
    n2 = hr2 // th2

    def level2(c, m):
        i0 = c * th2
        acc = jnp.zeros((th2 * wp2, 256), jnp.float32)
        for di in range(3):
            for dvi in range(3):
                win = f1p_ref[pl.ds(i0 + di, th2), pl.ds(dvi, wp2), :]
                patch = win.reshape(th2 * wp2, 512)
                acc = acc + jnp.dot(patch, w2_ref[di * 3 + dvi],
                                    preferred_element_type=jnp.float32)
        y = acc + b2_ref[...]
        y = y * _sigmoid(y)
        ybf = y.astype(jnp.bfloat16)
        z = jnp.dot(ybf, h2w_ref[...], preferred_element_type=jnp.float32)
        s = _sigmoid(z + h2b_ref[...])
        p = s[:, 0:64] * s[:, 64:128]
        m = jnp.maximum(m, jnp.max(p))
        yv = ybf.reshape(th2 // 2, 2, wp2 // 2, 2, 256)
        for par in range(2):
            for vp in range(2):
                f2p_ref[pl.ds(1 + i0 // 2, th2 // 2), pl.ds(1, wp2 // 2),
                        pl.ds(vp * 512 + par * 256, 256)] = yv[:, par, :, vp, :]
        return m

    m = jax.lax.fori_loop(0, n2, level2, m)

    th3 = min(16, hr3)
    for c in range(hr3 // th3):
        i0 = c * th3
        acc = jnp.zeros((th3 * wp3, 256), jnp.float32)
        for di in range(3):
            for dvi in range(3):
                win = f2p_ref[pl.ds(i0 + di, th3), pl.ds(dvi, wp3), :]
                patch = win.reshape(th3 * wp3, 1024)
                acc = acc + jnp.dot(patch, w3_ref[di * 3 + dvi],
                                    preferred_element_type=jnp.float32)
        y = acc + b3_ref[...]
        y = y * _sigmoid(y)
        ybf = y.astype(jnp.bfloat16)
        z = jnp.dot(ybf, h3w_ref[...], preferred_element_type=jnp.float32)
        s = _sigmoid(z + h3b_ref[...])
        p = s[:, 0:64] * s[:, 64:128]
        m = jnp.maximum(m, jnp.max(p))

    smax_ref[...] = m * jnp.ones_like(smax_ref)


def _loss_kernel(sm_ref, loss_ref):
    m = jnp.max(sm_ref[...])
    loss_ref[...] = -jnp.log(jnp.maximum(1.0 - m, 1e-9)) * jnp.ones_like(loss_ref)


def _space_to_depth(x):
    b, h, w, c = x.shape
    x = x.reshape(b, h // 2, 2, w // 2, 2, c)
    x = jnp.transpose(x, (0, 1, 3, 2, 4, 5))
    return x.reshape(b, h // 2, w // 2, 4 * c)


def kernel(x, attack_target, conv1_w, conv1_b, conv2_w, conv2_b, conv3_w,
           conv3_b, head1_w, head1_b, head2_w, head2_b, head3_w, head3_b):
    t = jnp.asarray(attack_target, jnp.int32)
    x = jnp.transpose(x, (0, 2, 3, 1)).astype(jnp.float32)
    bsz, h, w, _ = x.shape

    xs = _space_to_depth(x)
    xq = xs.reshape(bsz, h // 2, w // 8, 48)
    xq = jnp.pad(xq, ((0, 0), (1, 1), (1, 1), (0, 0))).astype(jnp.bfloat16)

    w1 = _quad_conv1_weights(conv1_w)
    w2 = _pair_conv_weights(conv2_w)
    w3 = _pair_conv_weights(conv3_w)
    b1 = jnp.tile(conv1_b, (1, 4))
    b2 = jnp.tile(conv2_b, (1, 2))
    b3 = jnp.tile(conv3_b, (1, 2))
    h1w, h1b = _head_select(head1_w, head1_b, t, 4)
    h2w, h2b = _head_select(head2_w, head2_b, t, 2)
    h3w, h3b = _head_select(head3_w, head3_b, t, 2)

    const = lambda bi: (0, 0)
    const3 = lambda bi: (0, 0, 0)
    sm = pl.pallas_call(
        functools.partial(_mega_kernel, h=h, w=w),
        grid=(bsz,),
        in_specs=[
            pl.BlockSpec((None, h // 2 + 2, w // 8 + 2, 48),
                         lambda bi: (bi, 0, 0, 0)),
            pl.BlockSpec((9, 48, 256), const3),
            pl.BlockSpec((1, 256), const),
            pl.BlockSpec((256, 128), const),
            pl.BlockSpec((1, 128), const),
            pl.BlockSpec((9, 512, 256), const3),
            pl.BlockSpec((1, 256), const),
            pl.BlockSpec((256, 128), const),
            pl.BlockSpec((1, 128), const),
            pl.BlockSpec((9, 1024, 256), const3),
            pl.BlockSpec((1, 256), const),
            pl.BlockSpec((256, 128), const),
            pl.BlockSpec((1, 128), const),
        ],
        out_specs=pl.BlockSpec((None, 1, 128), lambda bi: (bi, 0, 0)),
        out_shape=jax.ShapeDtypeStruct((bsz, 1, 128), jnp.float32),
        scratch_shapes=[
            pltpu.VMEM((h // 4 + 2, w // 8 + 2, 512), jnp.bfloat16),
            pltpu.VMEM((h // 8 + 2, w // 16 + 2, 1024), jnp.bfloat16),
        ],
        compiler_params=pltpu.CompilerParams(
            dimension_semantics=("parallel",),
            vmem_limit_bytes=VMEM_LIMIT),
    )(xq, w1, b1, h1w, h1b, w2, b2, h2w, h2b, w3, b3, h3w, h3b)

    loss = pl.pallas_call(
        _loss_kernel,
        grid=(1,),
        in_specs=[pl.BlockSpec((bsz, 128), lambda i: (0, 0))],
        out_specs=pl.BlockSpec((1, 1), lambda i: (0, 0)),
        out_shape=jax.ShapeDtypeStruct((1, 1), jnp.float32),
    )(sm.reshape(bsz, 128))
    return loss[0, 0]


# speedup vs baseline: 3.7818x; 1.1771x over previous
import functools
import math

import jax
import jax.numpy as jnp
from jax.experimental import pallas as pl
from jax.experimental.pallas import tpu as pltpu

NUM_CLASSES = 8
NUM_ANCHORS = 3
NC5 = 5 + NUM_CLASSES
VMEM_LIMIT = 100 * 1024 * 1024


def _quad_conv1_weights(w):
    zero = jnp.zeros((12, 64), w.dtype)
    taps = []
    for ki in range(3):
        for dvi in range(3):
            rows = []
            for vp in range(4):
                cols = []
                for q4 in range(4):
                    kj = 4 * (dvi - 1) + vp + 1 - q4
                    cols.append(w[ki * 3 + kj] if 0 <= kj < 3 else zero)
                rows.append(jnp.concatenate(cols, axis=1))
            taps.append(jnp.concatenate(rows, axis=0))
    return jnp.stack(taps)


def _pair_conv_weights(w):
    cin, cout = w.shape[1], w.shape[2]
    zero = jnp.zeros((cin, cout), w.dtype)
    taps = []
    for di in range(3):
        for dvi in range(3):
            rows = []
            for vp in range(2):
                cols = []
                for q in range(2):
                    dj = 2 * (dvi - 1) + vp + 1 - q
                    cols.append(w[di * 3 + dj] if 0 <= dj < 3 else zero)
                rows.append(jnp.concatenate(cols, axis=1))
            taps.append(jnp.concatenate(rows, axis=0))
    return jnp.stack(taps)


def _head_select(w, b, t, nq):
    cin = w.shape[0]
    obj_cols = jnp.array([a * NC5 + 4 for a in range(NUM_ANCHORS)], jnp.int32)
    cls_cols = jnp.array([a * NC5 + 5 for a in range(NUM_ANCHORS)], jnp.int32) + t
    wobj = jnp.take(w, obj_cols, axis=1).astype(jnp.bfloat16)
    wcls = jnp.take(w, cls_cols, axis=1).astype(jnp.bfloat16)
    ws = jnp.zeros((nq * cin, 128), jnp.bfloat16)
    bs = jnp.full((1, 128), -30.0, jnp.float32)
    for q in range(nq):
        ws = ws.at[q * cin:(q + 1) * cin, q * 16:q * 16 + 3].set(wobj)
        ws = ws.at[q * cin:(q + 1) * cin, 64 + q * 16:64 + q * 16 + 3].set(wcls)
        bs = bs.at[0, q * 16:q * 16 + 3].set(jnp.take(b[0], obj_cols))
        bs = bs.at[0, 64 + q * 16:64 + q * 16 + 3].set(jnp.take(b[0], cls_cols))
    return ws, bs


def _sigmoid(z):
    return 1.0 / (1.0 + jnp.exp(-z))


def _mega_kernel(xq_ref, w1_ref, b1_ref, h1w_ref, h1b_ref,
                 w2_ref, b2_ref, h2w_ref, h2b_ref,
                 w3_ref, b3_ref, h3w_ref, h3b_ref,
                 smax_ref, f1p_ref, f2p_ref,
                 *, h, w):
    hr1, wq1 = h // 2, w // 8
    hr2, wp2 = h // 4, w // 8
    hr3, wp3 = h // 8, w // 16

    f1p_ref[pl.ds(0, 1), :, :] = jnp.zeros((1, wp2 + 2, 512), jnp.bfloat16)
    f1p_ref[pl.ds(hr2 + 1, 1), :, :] = jnp.zeros((1, wp2 + 2, 512), jnp.bfloat16)
    f1p_ref[:, pl.ds(0, 1), :] = jnp.zeros((hr2 + 2, 1, 512), jnp.bfloat16)
    f1p_ref[:, pl.ds(wp2 + 1, 1), :] = jnp.zeros((hr2 + 2, 1, 512), jnp.bfloat16)
    f2p_ref[pl.ds(0, 1), :, :] = jnp.zeros((1, wp3 + 2, 1024), jnp.bfloat16)
    f2p_ref[pl.ds(hr3 + 1, 1), :, :] = jnp.zeros((1, wp3 + 2, 1024), jnp.bfloat16)
    f2p_ref[:, pl.ds(0, 1), :] = jnp.zeros((hr3 + 2, 1, 1024), jnp.bfloat16)
    f2p_ref[:, pl.ds(wp3 + 1, 1), :] = jnp.zeros((hr3 + 2, 1, 1024), jnp.bfloat16)

    th1 = min(32, hr1)
    n1 = hr1 // th1

    def level1(c, m):
        r0 = c * th1
        acc = jnp.zeros((th1 * wq1, 256), jnp.float32)
        for ki in range(3):
            for dvi in range(3):
                win = xq_ref[pl.ds(r0 + ki, th1), pl.ds(dvi, wq1), :]
                patch = win.reshape(th1 * wq1, 48)
                acc = acc + jnp.dot(patch, w1_ref[ki * 3 + dvi],
                                    preferred_element_type=jnp.float32)
        y = acc + b1_ref[...]
        y = y * _sigmoid(y)
        ybf = y.astype(jnp.bfloat16)
        z = jnp.dot(ybf, h1w_ref[...], preferred_element_type=jnp.float32)
        s = _sigmoid(z + h1b_ref[...])
        p = s[:, 0:64] * s[:, 64:128]
        m = jnp.maximum(m, jnp.max(p))
        yv = ybf.reshape(th1 // 2, 2, wq1, 256)
        for par in range(2):
            f1p_ref[pl.ds(1 + r0 // 2, th1 // 2), pl.ds(1, wq1),
                    pl.ds(par * 128, 128)] = yv[:, par, :, 0:128]
            f1p_ref[pl.ds(1 + r0 // 2, th1 // 2), pl.ds(1, wq1),
                    pl.ds(256 + par * 128, 128)] = yv[:, par, :, 128:256]
        return m

    m = jax.lax.fori_loop(0, n1, level1, jnp.zeros((), jnp.float32))

    th2 = min(32, hr2)
    n2 = hr2 // th2

    def level2(c, m):
        i0 = c * th2
        acc = jnp.zeros((th2 * wp2, 256), jnp.float32)
        for di in range(3):
            for dvi in range(3):
                win = f1p_ref[pl.ds(i0 + di, th2), pl.ds(dvi, wp2), :]
                patch = win.reshape(th2 * wp2, 512)
                acc = acc + jnp.dot(patch, w2_ref[di * 3 + dvi],
                                    preferred_element_type=jnp.float32)
        y = acc + b2_ref[...]
        y = y * _sigmoid(y)
        ybf = y.astype(jnp.bfloat16)
        z = jnp.dot(ybf, h2w_ref[...], preferred_element_type=jnp.float32)
        s = _sigmoid(z + h2b_ref[...])
        p = s[:, 0:64] * s[:, 64:128]
        m = jnp.maximum(m, jnp.max(p))
        yv = ybf.reshape(th2 // 2, 2, wp2 // 2, 2, 256)
        for par in range(2):
            for vp in range(2):
                f2p_ref[pl.ds(1 + i0 // 2, th2 // 2), pl.ds(1, wp2 // 2),
                        pl.ds(vp * 512 + par * 256, 256)] = yv[:, par, :, vp, :]
        return m

    m = jax.lax.fori_loop(0, n2, level2, m)

    th3 = min(32, hr3)
    for c in range(hr3 // th3):
        i0 = c * th3
        acc = jnp.zeros((th3 * wp3, 256), jnp.float32)
        for di in range(3):
            for dvi in range(3):
                win = f2p_ref[pl.ds(i0 + di, th3), pl.ds(dvi, wp3), :]
                patch = win.reshape(th3 * wp3, 1024)
                acc = acc + jnp.dot(patch, w3_ref[di * 3 + dvi],
                                    preferred_element_type=jnp.float32)
        y = acc + b3_ref[...]
        y = y * _sigmoid(y)
        ybf = y.astype(jnp.bfloat16)
        z = jnp.dot(ybf, h3w_ref[...], preferred_element_type=jnp.float32)
        s = _sigmoid(z + h3b_ref[...])
        p = s[:, 0:64] * s[:, 64:128]
        m = jnp.maximum(m, jnp.max(p))

    smax_ref[...] = m * jnp.ones_like(smax_ref)


def _loss_kernel(sm_ref, loss_ref):
    m = jnp.max(sm_ref[...])
    loss_ref[...] = -jnp.log(jnp.maximum(1.0 - m, 1e-9)) * jnp.ones_like(loss_ref)


def _space_to_depth(x):
    b, h, w, c = x.shape
    x = x.reshape(b, h // 2, 2, w // 2, 2, c)
    x = jnp.transpose(x, (0, 1, 3, 2, 4, 5))
    return x.reshape(b, h // 2, w // 2, 4 * c)


def kernel(x, attack_target, conv1_w, conv1_b, conv2_w, conv2_b, conv3_w,
           conv3_b, head1_w, head1_b, head2_w, head2_b, head3_w, head3_b):
    t = jnp.asarray(attack_target, jnp.int32)
    x = jnp.transpose(x, (0, 2, 3, 1)).astype(jnp.float32)
    bsz, h, w, _ = x.shape

    xs = _space_to_depth(x)
    xq = xs.reshape(bsz, h // 2, w // 8, 48)
    xq = jnp.pad(xq, ((0, 0), (1, 1), (1, 1), (0, 0))).astype(jnp.bfloat16)

    w1 = _quad_conv1_weights(conv1_w)
    w2 = _pair_conv_weights(conv2_w)
    w3 = _pair_conv_weights(conv3_w)
    b1 = jnp.tile(conv1_b, (1, 4))
    b2 = jnp.tile(conv2_b, (1, 2))
    b3 = jnp.tile(conv3_b, (1, 2))
    h1w, h1b = _head_select(head1_w, head1_b, t, 4)
    h2w, h2b = _head_select(head2_w, head2_b, t, 2)
    h3w, h3b = _head_select(head3_w, head3_b, t, 2)

    const = lambda bi: (0, 0)
    const3 = lambda bi: (0, 0, 0)
    sm = pl.pallas_call(
        functools.partial(_mega_kernel, h=h, w=w),
        grid=(bsz,),
        in_specs=[
            pl.BlockSpec((None, h // 2 + 2, w // 8 + 2, 48),
                         lambda bi: (bi, 0, 0, 0)),
            pl.BlockSpec((9, 48, 256), const3),
            pl.BlockSpec((1, 256), const),
            pl.BlockSpec((256, 128), const),
            pl.BlockSpec((1, 128), const),
            pl.BlockSpec((9, 512, 256), const3),
            pl.BlockSpec((1, 256), const),
            pl.BlockSpec((256, 128), const),
            pl.BlockSpec((1, 128), const),
            pl.BlockSpec((9, 1024, 256), const3),
            pl.BlockSpec((1, 256), const),
            pl.BlockSpec((256, 128), const),
            pl.BlockSpec((1, 128), const),
        ],
        out_specs=pl.BlockSpec((None, 1, 128), lambda bi: (bi, 0, 0)),
        out_shape=jax.ShapeDtypeStruct((bsz, 1, 128), jnp.float32),
        scratch_shapes=[
            pltpu.VMEM((h // 4 + 2, w // 8 + 2, 512), jnp.bfloat16),
            pltpu.VMEM((h // 8 + 2, w // 16 + 2, 1024), jnp.bfloat16),
        ],
        compiler_params=pltpu.CompilerParams(
            dimension_semantics=("parallel",),
            vmem_limit_bytes=VMEM_LIMIT),
    )(xq, w1, b1, h1w, h1b, w2, b2, h2w, h2b, w3, b3, h3w, h3b)

    loss = pl.pallas_call(
        _loss_kernel,
        grid=(1,),
        in_specs=[pl.BlockSpec((bsz, 128), lambda i: (0, 0))],
        out_specs=pl.BlockSpec((1, 1), lambda i: (0, 0)),
        out_shape=jax.ShapeDtypeStruct((1, 1), jnp.float32),
    )(sm.reshape(bsz, 128))
    return loss[0, 0]
